# baseline (speedup 1.0000x reference)
import functools

import jax
import jax.numpy as jnp
from jax import lax
from jax.experimental import pallas as pl
from jax.experimental.pallas import tpu as pltpu
from jax.experimental.pallas import tpu_sc as plsc

_NC = 2
_NS = 16
_NW = _NC * _NS
_LANES = 16


@functools.cache
def _build(B, n_cond, dim, n_cat):
    total = B * n_cond
    per_w = total // _NW
    chunk = 1664 if per_w % 1664 == 0 else per_w
    n_chunks = per_w // chunk
    mesh = plsc.VectorSubcoreMesh(core_axis_name="c", subcore_axis_name="s")

    @functools.partial(
        pl.kernel,
        out_type=jax.ShapeDtypeStruct((total, dim), jnp.float32),
        mesh=mesh,
        scratch_types=[
            pltpu.VMEM((chunk,), jnp.int32),
            pltpu.VMEM((chunk,), jnp.int32),
            pltpu.VMEM((chunk,), jnp.int32),
            pltpu.VMEM((chunk, dim), jnp.float32),
            pltpu.SemaphoreType.DMA,
        ],
        compiler_params=pltpu.CompilerParams(use_tc_tiling_on_sc=False),
    )
    def run(ids_hbm, cond_hbm, cat_hbm, offs_hbm, cidx_hbm, out_hbm,
            idx_v, offs_v, cidx_v, rows_v, sem):
        wid = lax.axis_index("s") * _NC + lax.axis_index("c")
        base = wid * per_w
        pltpu.sync_copy(offs_hbm, offs_v)
        pltpu.sync_copy(cidx_hbm, cidx_v)

        def chunk_body(i, carry):
            cbase = base + i * chunk
            pltpu.sync_copy(ids_hbm.at[pl.ds(cbase, chunk)], idx_v)

            def add_body(k, c):
                sl = pl.ds(k * _LANES, _LANES)
                idx_v[sl] = idx_v[sl] + offs_v[sl]
                return c

            lax.fori_loop(0, chunk // _LANES, add_body, 0)
            pltpu.async_copy(cond_hbm.at[cidx_v], rows_v, sem).wait()
            pltpu.async_copy(cat_hbm.at[idx_v], rows_v, sem, add=True).wait()
            pltpu.sync_copy(rows_v, out_hbm.at[pl.ds(cbase, chunk)])
            return carry

        lax.fori_loop(0, n_chunks, chunk_body, 0)

    return run, chunk


def kernel(cat_ids, cond_table, cat_table):
    B, n_cond = cat_ids.shape
    dim = cat_table.shape[1]
    n_cat = cat_table.shape[0] // n_cond
    run, chunk = _build(B, n_cond, dim, n_cat)
    ids_flat = cat_ids.reshape(-1).astype(jnp.int32)
    reps = chunk // n_cond
    offs = jnp.tile(jnp.arange(n_cond, dtype=jnp.int32) * n_cat, reps)
    cidx = jnp.tile(jnp.arange(1, n_cond + 1, dtype=jnp.int32), reps)
    out = run(ids_flat, cond_table, cat_table, offs, cidx)
    return out.reshape(B, n_cond, dim)


# speedup vs baseline: 1.3419x; 1.3419x over previous
import functools

import jax
import jax.numpy as jnp
from jax import lax
from jax.experimental import pallas as pl
from jax.experimental.pallas import tpu as pltpu
from jax.experimental.pallas import tpu_sc as plsc

_NC = 2
_NS = 16
_NW = _NC * _NS
_L = 16


@functools.cache
def _build(B, n_cond, dim, n_cat):
    total = B * n_cond
    per_w = total // _NW
    chunk = 1664 if per_w % 1664 == 0 else per_w
    n_chunks = per_w // chunk
    reps = chunk // n_cond
    mesh = plsc.VectorSubcoreMesh(core_axis_name="c", subcore_axis_name="s")

    @functools.partial(
        pl.kernel,
        out_type=jax.ShapeDtypeStruct((total, dim), jnp.float32),
        mesh=mesh,
        scratch_types=[
            pltpu.VMEM((chunk,), jnp.int32),
            pltpu.VMEM((chunk,), jnp.int32),
            pltpu.VMEM((chunk,), jnp.int32),
            pltpu.VMEM((chunk,), jnp.int32),
            pltpu.VMEM((chunk, dim), jnp.float32),
            pltpu.VMEM((chunk, dim), jnp.float32),
            pltpu.VMEM((chunk,), jnp.int32),
            pltpu.VMEM((n_cond, dim), jnp.float32),
            pltpu.SemaphoreType.DMA,
            pltpu.SemaphoreType.DMA,
            pltpu.SemaphoreType.DMA,
            pltpu.SemaphoreType.DMA,
            pltpu.SemaphoreType.DMA,
            pltpu.SemaphoreType.DMA,
            pltpu.SemaphoreType.DMA,
            pltpu.SemaphoreType.DMA,
        ],
        compiler_params=pltpu.CompilerParams(use_tc_tiling_on_sc=False),
    )
    def run(ids_hbm, cond_hbm, cat_hbm, offs_hbm, out_hbm,
            ix0, ix1, ix2, ix3, rw0, rw1, offs_v, bias_v,
            si0, si1, si2, si3, sg0, sg1, so0, so1):
        wid = lax.axis_index("s") * _NC + lax.axis_index("c")
        base = wid * per_w
        idxs = [ix0, ix1, ix2, ix3]
        rows = [rw0, rw1]
        sid = [si0, si1, si2, si3]
        sg = [sg0, sg1]
        so = [so0, so1]
        pltpu.sync_copy(offs_hbm, offs_v)
        pltpu.sync_copy(cond_hbm.at[pl.ds(1, n_cond)], bias_v)

        descs = {}

        def s_ids(i):
            descs["ids", i] = pltpu.async_copy(
                ids_hbm.at[pl.ds(base + i * chunk, chunk)], idxs[i % 4],
                sid[i % 4])

        def s_gat(i):
            descs["gat", i] = pltpu.async_copy(
                cat_hbm.at[idxs[i % 4]], rows[i % 2], sg[i % 2])

        def s_out(i):
            descs["out", i] = pltpu.async_copy(
                rows[i % 2], out_hbm.at[pl.ds(base + i * chunk, chunk)],
                so[i % 2])

        def add_offs(i):
            ix = idxs[i % 4]

            def body(k, c):
                sl = pl.ds(k * _L, _L)
                ix[sl] = ix[sl] + offs_v[sl]
                return c

            lax.fori_loop(0, chunk // _L, body, 0)

        def add_bias(i):
            r = rows[i % 2]

            def body(rep, c):
                r0 = rep * n_cond
                for rb in range(n_cond):
                    for h in range(dim // _L):
                        sl = pl.ds(h * _L, _L)
                        r[r0 + rb, sl] = r[r0 + rb, sl] + bias_v[rb, sl]
                return c

            lax.fori_loop(0, reps, body, 0)

        s_ids(0)
        s_ids(1)
        descs["ids", 0].wait()
        add_offs(0)
        s_gat(0)
        if n_chunks > 2:
            s_ids(2)
        for i in range(n_chunks):
            if i + 1 < n_chunks:
                descs["ids", i + 1].wait()
                add_offs(i + 1)
                if i >= 1:
                    descs["out", i - 1].wait()
                s_gat(i + 1)
                if i + 3 < n_chunks:
                    s_ids(i + 3)
            descs["gat", i].wait()
            add_bias(i)
            s_out(i)
        if n_chunks >= 2:
            descs["out", n_chunks - 2].wait()
        descs["out", n_chunks - 1].wait()

    return run, chunk


def kernel(cat_ids, cond_table, cat_table):
    B, n_cond = cat_ids.shape
    dim = cat_table.shape[1]
    n_cat = cat_table.shape[0] // n_cond
    run, chunk = _build(B, n_cond, dim, n_cat)
    ids_flat = cat_ids.reshape(-1).astype(jnp.int32)
    offs = jnp.tile(jnp.arange(n_cond, dtype=jnp.int32) * n_cat,
                    chunk // n_cond)
    out = run(ids_flat, cond_table, cat_table, offs)
    return out.reshape(B, n_cond, dim)
